# baseline (speedup 1.0000x reference)
import functools

import jax
import jax.numpy as jnp
from jax import lax
from jax.experimental import pallas as pl
from jax.experimental.pallas import tpu as pltpu
from jax.experimental.pallas import tpu_sc as plsc

_EPS = 1e-5


def _cdiv(a, b):
    return -(-a // b)


def _rup(a, b):
    return _cdiv(a, b) * b


def _ln(o):
    mu = jnp.mean(o, axis=-1, keepdims=True)
    var = jnp.mean(jnp.square(o - mu), axis=-1, keepdims=True)
    return (o - mu) * lax.rsqrt(var + _EPS)


def _row_spec(br, d):
    return pl.BlockSpec((br, d), lambda i: (i, 0))


def _w_spec(shape):
    return pl.BlockSpec(shape, lambda i: tuple(0 for _ in shape))


def _mlp_ln_tc(x, p):
    n, din = x.shape
    dh = p["W1"].shape[1]
    dout = p["W2"].shape[1]
    br = min(1024, _rup(n, 8))
    grid = _cdiv(n, br)

    def body(x_ref, w1, b1, w2, b2, o_ref):
        h = jax.nn.silu(x_ref[...] @ w1[...] + b1[...])
        o_ref[...] = _ln(h @ w2[...] + b2[...])

    return pl.pallas_call(
        body,
        grid=(grid,),
        in_specs=[_row_spec(br, din), _w_spec((din, dh)), _w_spec((1, dh)),
                  _w_spec((dh, dout)), _w_spec((1, dout))],
        out_specs=_row_spec(br, dout),
        out_shape=jax.ShapeDtypeStruct((n, dout), jnp.float32),
    )(x, p["W1"], p["b1"].reshape(1, dh), p["W2"], p["b2"].reshape(1, dout))


def _edge_tc(ag, bg, e, p):
    n, h = e.shape
    br = min(1024, _rup(n, 8))
    grid = _cdiv(n, br)
    w1 = p["W1"]
    w1s, w1d, w1e = w1[:h], w1[h:2 * h], w1[2 * h:]

    def body(a_ref, b_ref, e_ref, ws, wd, we, b1, w2, b2, msg_ref, enew_ref):
        h1 = jax.nn.silu(a_ref[...] @ ws[...] + b_ref[...] @ wd[...]
                         + e_ref[...] @ we[...] + b1[...])
        m = _ln(h1 @ w2[...] + b2[...])
        msg_ref[...] = m
        enew_ref[...] = e_ref[...] + m

    return pl.pallas_call(
        body,
        grid=(grid,),
        in_specs=[_row_spec(br, h), _row_spec(br, h), _row_spec(br, h),
                  _w_spec((h, h)), _w_spec((h, h)), _w_spec((h, h)),
                  _w_spec((1, h)), _w_spec((h, h)), _w_spec((1, h))],
        out_specs=[_row_spec(br, h), _row_spec(br, h)],
        out_shape=[jax.ShapeDtypeStruct((n, h), jnp.float32),
                   jax.ShapeDtypeStruct((n, h), jnp.float32)],
    )(ag, bg, e, w1s, w1d, w1e, p["b1"].reshape(1, h), p["W2"],
      p["b2"].reshape(1, h))


def _node_tc(xd, agg2, p, extra=None):
    n, h = xd.shape
    br = min(1024, _rup(n, 8))
    grid = _cdiv(n, br)
    v1 = p["W1"]
    v1a, v1b = v1[:h], v1[h:]

    def body_base(x_ref, g_ref, wa, wb, b1, w2, b2):
        agg = g_ref[0] + g_ref[1]
        h1 = jax.nn.silu(x_ref[...] @ wa[...] + agg @ wb[...] + b1[...])
        return x_ref[...] + _ln(h1 @ w2[...] + b2[...])

    if extra is None:
        def body(x_ref, g_ref, wa, wb, b1, w2, b2, o_ref):
            o_ref[...] = body_base(x_ref, g_ref, wa, wb, b1, w2, b2)
        extra_in = []
        extra_spec = []
    else:
        def body(x_ref, g_ref, wa, wb, b1, w2, b2, r_ref, o_ref):
            o_ref[...] = body_base(x_ref, g_ref, wa, wb, b1, w2, b2) + r_ref[...]
        extra_in = [extra]
        extra_spec = [_row_spec(br, h)]

    g_spec = pl.BlockSpec((2, br, h), lambda i: (0, i, 0))
    return pl.pallas_call(
        body,
        grid=(grid,),
        in_specs=[_row_spec(br, h), g_spec, _w_spec((h, h)), _w_spec((h, h)),
                  _w_spec((1, h)), _w_spec((h, h)), _w_spec((1, h))]
                 + extra_spec,
        out_specs=_row_spec(br, h),
        out_shape=jax.ShapeDtypeStruct((n, h), jnp.float32),
    )(xd, agg2, v1a, v1b, p["b1"].reshape(1, h), p["W2"],
      p["b2"].reshape(1, h), *extra_in)


def _final_tc(x30, w1, b1, w2, bs):
    n, h = x30.shape
    per = n // bs
    dh = w1.shape[1]
    sel = (jnp.arange(n)[None, :] // per
           == jnp.arange(bs)[:, None]).astype(jnp.float32) / per

    def body(x_ref, w1_ref, b1_ref, w2_ref, sel_ref, o_ref):
        hh = jax.nn.silu(x_ref[...] @ w1_ref[...] + b1_ref[...])
        o_ref[...] = sel_ref[...] @ (hh @ w2_ref[...])

    return pl.pallas_call(
        body,
        grid=(1,),
        in_specs=[_row_spec(n, h), _w_spec((h, dh)), _w_spec((1, dh)),
                  _w_spec((dh, 1)), _w_spec((bs, n))],
        out_specs=pl.BlockSpec((bs, 1), lambda i: (0, 0)),
        out_shape=jax.ShapeDtypeStruct((bs, 1), jnp.float32),
    )(x30, w1, b1.reshape(1, dh), w2, sel)


_NW = 32
_CH = 128


def _pipe_k(h):
    return 4 if h <= 64 else 2


def _sc_gather2(a, b, si, di):
    ne = si.shape[0]
    h = a.shape[1]
    k_n = _pipe_k(h)
    sup = k_n * _CH
    pw = ne // _NW
    steps = pw // sup
    mesh = plsc.VectorSubcoreMesh(core_axis_name="c", subcore_axis_name="s")

    @functools.partial(
        pl.kernel, mesh=mesh,
        compiler_params=pltpu.CompilerParams(use_tc_tiling_on_sc=False),
        out_type=(jax.ShapeDtypeStruct((ne, h), jnp.float32),
                  jax.ShapeDtypeStruct((ne, h), jnp.float32)),
        scratch_types=(pltpu.VMEM((2, sup), jnp.int32),
                       pltpu.VMEM((2, sup), jnp.int32),
                       pltpu.VMEM((sup, h), jnp.float32),
                       pltpu.VMEM((sup, h), jnp.float32),
                       pltpu.SemaphoreType.DMA, pltpu.SemaphoreType.DMA,
                       pltpu.SemaphoreType.DMA, pltpu.SemaphoreType.DMA,
                       pltpu.SemaphoreType.DMA, pltpu.SemaphoreType.DMA))
    def gk(a_h, b_h, s_h, d_h, ga_h, gb_h, si_v, di_v, ra_v, rb_v,
           sia, sib, sga, sgb, soa, sob):
        wid = lax.axis_index("s") * 2 + lax.axis_index("c")
        base = wid * pw
        pltpu.async_copy(s_h.at[pl.ds(base, sup)], si_v.at[0], sia)
        pltpu.async_copy(d_h.at[pl.ds(base, sup)], di_v.at[0], sib)

        def body(j, carry):
            slot = lax.rem(j, 2)
            r0 = base + j * sup
            pltpu.make_async_copy(s_h.at[pl.ds(0, sup)], si_v.at[slot],
                                  sia).wait()
            pltpu.make_async_copy(d_h.at[pl.ds(0, sup)], di_v.at[slot],
                                  sib).wait()
            nxt = jnp.where(j + 1 < steps, base + (j + 1) * sup, base)
            pltpu.async_copy(s_h.at[pl.ds(nxt, sup)], si_v.at[1 - slot], sia)
            pltpu.async_copy(d_h.at[pl.ds(nxt, sup)], di_v.at[1 - slot], sib)
            gas, gbs = [], []
            for k in range(k_n):
                sl = pl.ds(k * _CH, _CH)
                gas.append(pltpu.async_copy(a_h.at[si_v.at[slot, sl]],
                                            ra_v.at[sl], sga))
                gbs.append(pltpu.async_copy(b_h.at[di_v.at[slot, sl]],
                                            rb_v.at[sl], sgb))
            sts = []
            for k in range(k_n):
                sl = pl.ds(k * _CH, _CH)
                out_sl = pl.ds(r0 + k * _CH, _CH)
                gas[k].wait()
                sts.append(pltpu.async_copy(ra_v.at[sl], ga_h.at[out_sl], soa))
                gbs[k].wait()
                sts.append(pltpu.async_copy(rb_v.at[sl], gb_h.at[out_sl], sob))
            for dsc in sts:
                dsc.wait()
            return carry

        lax.fori_loop(0, steps, body, 0)
        fslot = steps % 2
        pltpu.make_async_copy(s_h.at[pl.ds(0, sup)], si_v.at[fslot],
                              sia).wait()
        pltpu.make_async_copy(d_h.at[pl.ds(0, sup)], di_v.at[fslot],
                              sib).wait()

    return gk(a, b, si, di)


def _sc_scatter_add(msg, di2, nd1):
    ne, h = msg.shape
    k_n = _pipe_k(h)
    sup = k_n * _CH
    pw = ne // _NW
    steps = pw // sup
    zr = nd1 // 16
    zsteps = zr // _CH
    zeros_blk = jnp.zeros((_CH, h), jnp.float32)
    mesh = plsc.VectorSubcoreMesh(core_axis_name="c", subcore_axis_name="s")

    @functools.partial(
        pl.kernel, mesh=mesh,
        compiler_params=pltpu.CompilerParams(use_tc_tiling_on_sc=False),
        out_type=jax.ShapeDtypeStruct((2, nd1, h), jnp.float32),
        scratch_types=(pltpu.VMEM((2, k_n, _CH), jnp.int32),
                       pltpu.VMEM((sup, h), jnp.float32),
                       pltpu.VMEM((_CH, h), jnp.float32),
                       pltpu.VMEM_SHARED((nd1, h), jnp.float32),
                       pltpu.SemaphoreType.DMA, pltpu.SemaphoreType.DMA,
                       pltpu.SemaphoreType.DMA, pltpu.SemaphoreType.DMA))
    def sk(m_h, d2_h, z_h, out_h, di_v, rm_v, zb_v, agg_s, sdi, sm, ssc, sz):
        cid = lax.axis_index("c")
        sid = lax.axis_index("s")
        wid = sid * 2 + cid
        pltpu.sync_copy(z_h, zb_v)
        zds = [pltpu.async_copy(zb_v, agg_s.at[pl.ds(sid * zr + t * _CH, _CH)],
                                sz) for t in range(zsteps)]
        for dsc in zds:
            dsc.wait()
        plsc.subcore_barrier()
        base_c = wid * (pw // _CH)
        pltpu.async_copy(d2_h.at[pl.ds(base_c, k_n)], di_v.at[0], sdi)

        def body(j, carry):
            slot = lax.rem(j, 2)
            pltpu.make_async_copy(d2_h.at[pl.ds(0, k_n)], di_v.at[slot],
                                  sdi).wait()
            nxt = jnp.where(j + 1 < steps, base_c + (j + 1) * k_n, base_c)
            pltpu.async_copy(d2_h.at[pl.ds(nxt, k_n)], di_v.at[1 - slot], sdi)
            r0 = wid * pw + j * sup
            lds = [pltpu.async_copy(m_h.at[pl.ds(r0 + k * _CH, _CH)],
                                    rm_v.at[pl.ds(k * _CH, _CH)], sm)
                   for k in range(k_n)]
            scs = []
            for k in range(k_n):
                lds[k].wait()
                scs.append(pltpu.async_copy(rm_v.at[pl.ds(k * _CH, _CH)],
                                            agg_s.at[di_v.at[slot, k]],
                                            ssc, add=True))
            for dsc in scs:
                dsc.wait()
            return carry

        lax.fori_loop(0, steps, body, 0)
        fslot = steps % 2
        pltpu.make_async_copy(d2_h.at[pl.ds(0, k_n)], di_v.at[fslot],
                              sdi).wait()
        plsc.subcore_barrier()
        ods = [pltpu.async_copy(agg_s.at[pl.ds(sid * zr + t * _CH, _CH)],
                                out_h.at[cid, pl.ds(sid * zr + t * _CH, _CH)],
                                sz) for t in range(zsteps)]
        for dsc in ods:
            dsc.wait()

    return sk(msg, di2, zeros_blk)


def _prep_edges(ei, ns_u, nd_u, bs, h):
    src, dst = ei[0], ei[1]
    srcb = jnp.concatenate([src + i * ns_u for i in range(bs)])
    dstb = jnp.concatenate([dst + i * nd_u for i in range(bs)])
    ne = srcb.shape[0]
    ne_pad = _rup(ne, _NW * _CH * _pipe_k(h))
    pad = ne_pad - ne
    si = jnp.pad(srcb, (0, pad))
    dg = jnp.pad(dstb, (0, pad))
    ds2 = jnp.pad(dstb, (0, pad),
                  constant_values=nd_u * bs).reshape(ne_pad // _CH, _CH)
    nd1 = _rup(nd_u * bs + 1, 16 * _CH)
    return si, dg, ds2, ne_pad, nd1


def _pad_rows(x, n):
    return jnp.pad(x, ((0, n - x.shape[0]), (0, 0)))


def _block_k(blk, xs_tab, xd_tab, e, si, dg, ds, nd, nd1, extra=None):
    ag, bg = _sc_gather2(xs_tab, xd_tab, si, dg)
    msg, e_new = _edge_tc(ag, bg, e, blk["edge_mlp"])
    agg2 = _sc_scatter_add(msg, ds, nd1)
    xd_new = _node_tc(xd_tab, agg2[:, :nd, :], blk["node_mlp"], extra)
    return xd_new, e_new


def _hop(mp, pp, x_src, sname, dname, geo, edges, bs):
    nd_u = geo[dname + "_latlons"].shape[0]
    ns_u = geo[sname + "_latlons"].shape[0]
    nd = nd_u * bs

    xs = _mlp_ln_tc(x_src, mp["emb_src"])
    xd = jnp.tile(_mlp_ln_tc(geo[dname + "_latlons"], mp["emb_dst"]), (bs, 1))
    e = jnp.tile(_mlp_ln_tc(geo[sname + "_" + dname + "_edge_attr"],
                            mp["emb_edge"]), (bs, 1))
    h = xd.shape[1]
    si, dg, ds, ne_pad, nd1 = _prep_edges(
        edges[sname + "_" + dname + "_edge_index"], ns_u, nd_u, bs, h)
    e = _pad_rows(e, ne_pad)
    xd, _ = _block_k(mp["blocks"][0], xs, xd, e, si, dg, ds, nd, nd1)
    x_lat = xd

    ep = jnp.tile(_mlp_ln_tc(geo[dname + "_" + dname + "_edge_attr"],
                             pp["emb_edge"]), (bs, 1))
    sip, dgp, dsp, nep_pad, nd1p = _prep_edges(
        edges[dname + "_" + dname + "_edge_index"], nd_u, nd_u, bs, h)
    ep = _pad_rows(ep, nep_pad)
    xp = x_lat
    nblk = len(pp["blocks"])
    for i, blk in enumerate(pp["blocks"]):
        res = x_lat if i == nblk - 1 else None
        xp, ep = _block_k(blk, xp, xp, ep, sip, dgp, dsp, nd, nd1p, extra=res)
    return xp


def kernel(x, params, geo, edges):
    bs, n_era, in_ch = x.shape
    xf = jnp.concatenate(
        [x.reshape(bs * n_era, in_ch), jnp.tile(geo["era_latlons"], (bs, 1))],
        axis=-1)
    cur = xf
    for s, d in (("era", "h33"), ("h33", "h32"), ("h32", "h31"),
                 ("h31", "h30")):
        cur = _hop(params[d + "_mapper"], params[d + "_proc"], cur, s, d,
                   geo, edges, bs)
    return _final_tc(cur, params["final_W1"], params["final_b1"],
                     params["final_W2"], bs)


# speedup vs baseline: 1.1252x; 1.1252x over previous
import functools

import jax
import jax.numpy as jnp
from jax import lax
from jax.experimental import pallas as pl
from jax.experimental.pallas import tpu as pltpu
from jax.experimental.pallas import tpu_sc as plsc

_EPS = 1e-5


def _cdiv(a, b):
    return -(-a // b)


def _rup(a, b):
    return _cdiv(a, b) * b


def _ln(o):
    mu = jnp.mean(o, axis=-1, keepdims=True)
    var = jnp.mean(jnp.square(o - mu), axis=-1, keepdims=True)
    return (o - mu) * lax.rsqrt(var + _EPS)


def _row_spec(br, d):
    return pl.BlockSpec((br, d), lambda i: (i, 0))


def _w_spec(shape):
    return pl.BlockSpec(shape, lambda i: tuple(0 for _ in shape))


def _mlp_ln_tc(x, p):
    n, din = x.shape
    dh = p["W1"].shape[1]
    dout = p["W2"].shape[1]
    br = min(1024, _rup(n, 8))
    grid = _cdiv(n, br)

    def body(x_ref, w1, b1, w2, b2, o_ref):
        h = jax.nn.silu(x_ref[...] @ w1[...] + b1[...])
        o_ref[...] = _ln(h @ w2[...] + b2[...])

    return pl.pallas_call(
        body,
        grid=(grid,),
        in_specs=[_row_spec(br, din), _w_spec((din, dh)), _w_spec((1, dh)),
                  _w_spec((dh, dout)), _w_spec((1, dout))],
        out_specs=_row_spec(br, dout),
        out_shape=jax.ShapeDtypeStruct((n, dout), jnp.float32),
    )(x, p["W1"], p["b1"].reshape(1, dh), p["W2"], p["b2"].reshape(1, dout))


def _edge_tc(ag, bg, e, p):
    n, h = e.shape
    br = min(1024, _rup(n, 8))
    grid = _cdiv(n, br)
    w1 = p["W1"]
    w1s, w1d, w1e = w1[:h], w1[h:2 * h], w1[2 * h:]

    def body(a_ref, b_ref, e_ref, ws, wd, we, b1, w2, b2, msg_ref, enew_ref):
        h1 = jax.nn.silu(a_ref[...] @ ws[...] + b_ref[...] @ wd[...]
                         + e_ref[...] @ we[...] + b1[...])
        m = _ln(h1 @ w2[...] + b2[...])
        msg_ref[...] = m
        enew_ref[...] = e_ref[...] + m

    return pl.pallas_call(
        body,
        grid=(grid,),
        in_specs=[_row_spec(br, h), _row_spec(br, h), _row_spec(br, h),
                  _w_spec((h, h)), _w_spec((h, h)), _w_spec((h, h)),
                  _w_spec((1, h)), _w_spec((h, h)), _w_spec((1, h))],
        out_specs=[_row_spec(br, h), _row_spec(br, h)],
        out_shape=[jax.ShapeDtypeStruct((n, h), jnp.float32),
                   jax.ShapeDtypeStruct((n, h), jnp.float32)],
    )(ag, bg, e, w1s, w1d, w1e, p["b1"].reshape(1, h), p["W2"],
      p["b2"].reshape(1, h))


def _node_tc(xd, agg2, p, extra=None):
    n, h = xd.shape
    br = min(1024, _rup(n, 8))
    grid = _cdiv(n, br)
    v1 = p["W1"]
    v1a, v1b = v1[:h], v1[h:]

    def body_base(x_ref, g_ref, wa, wb, b1, w2, b2):
        agg = g_ref[0] + g_ref[1]
        h1 = jax.nn.silu(x_ref[...] @ wa[...] + agg @ wb[...] + b1[...])
        return x_ref[...] + _ln(h1 @ w2[...] + b2[...])

    if extra is None:
        def body(x_ref, g_ref, wa, wb, b1, w2, b2, o_ref):
            o_ref[...] = body_base(x_ref, g_ref, wa, wb, b1, w2, b2)
        extra_in = []
        extra_spec = []
    else:
        def body(x_ref, g_ref, wa, wb, b1, w2, b2, r_ref, o_ref):
            o_ref[...] = body_base(x_ref, g_ref, wa, wb, b1, w2, b2) + r_ref[...]
        extra_in = [extra]
        extra_spec = [_row_spec(br, h)]

    g_spec = pl.BlockSpec((2, br, h), lambda i: (0, i, 0))
    return pl.pallas_call(
        body,
        grid=(grid,),
        in_specs=[_row_spec(br, h), g_spec, _w_spec((h, h)), _w_spec((h, h)),
                  _w_spec((1, h)), _w_spec((h, h)), _w_spec((1, h))]
                 + extra_spec,
        out_specs=_row_spec(br, h),
        out_shape=jax.ShapeDtypeStruct((n, h), jnp.float32),
    )(xd, agg2, v1a, v1b, p["b1"].reshape(1, h), p["W2"],
      p["b2"].reshape(1, h), *extra_in)


def _final_tc(x30, w1, b1, w2, bs):
    n, h = x30.shape
    per = n // bs
    dh = w1.shape[1]
    sel = (jnp.arange(n)[None, :] // per
           == jnp.arange(bs)[:, None]).astype(jnp.float32) / per

    def body(x_ref, w1_ref, b1_ref, w2_ref, sel_ref, o_ref):
        hh = jax.nn.silu(x_ref[...] @ w1_ref[...] + b1_ref[...])
        o_ref[...] = sel_ref[...] @ (hh @ w2_ref[...])

    return pl.pallas_call(
        body,
        grid=(1,),
        in_specs=[_row_spec(n, h), _w_spec((h, dh)), _w_spec((1, dh)),
                  _w_spec((dh, 1)), _w_spec((bs, n))],
        out_specs=pl.BlockSpec((bs, 1), lambda i: (0, 0)),
        out_shape=jax.ShapeDtypeStruct((bs, 1), jnp.float32),
    )(x30, w1, b1.reshape(1, dh), w2, sel)


_NW = 32
_CH = 128


def _sc_gather2(a, b, si, di):
    ne = si.shape[0]
    h = a.shape[1]
    pw = ne // _NW
    pairs = pw // (2 * _CH)
    mesh = plsc.VectorSubcoreMesh(core_axis_name="c", subcore_axis_name="s")

    @functools.partial(
        pl.kernel, mesh=mesh,
        compiler_params=pltpu.CompilerParams(use_tc_tiling_on_sc=False),
        out_type=(jax.ShapeDtypeStruct((ne, h), jnp.float32),
                  jax.ShapeDtypeStruct((ne, h), jnp.float32)),
        scratch_types=(pltpu.VMEM((_CH,), jnp.int32),
                       pltpu.VMEM((_CH,), jnp.int32),
                       pltpu.VMEM((_CH,), jnp.int32),
                       pltpu.VMEM((_CH,), jnp.int32),
                       pltpu.VMEM((_CH, h), jnp.float32),
                       pltpu.VMEM((_CH, h), jnp.float32),
                       pltpu.VMEM((_CH, h), jnp.float32),
                       pltpu.VMEM((_CH, h), jnp.float32),
                       pltpu.SemaphoreType.DMA, pltpu.SemaphoreType.DMA,
                       pltpu.SemaphoreType.DMA, pltpu.SemaphoreType.DMA))
    def gk(a_h, b_h, s_h, d_h, ga_h, gb_h,
           siA, diA, siB, diB, raA, rbA, raB, rbB, sga, sgb, soa, sob):
        wid = lax.axis_index("s") * 2 + lax.axis_index("c")
        base = wid * pw

        def body(p, carry):
            r0 = base + p * 2 * _CH
            r1 = r0 + _CH
            pltpu.sync_copy(s_h.at[pl.ds(r0, _CH)], siA)
            pltpu.sync_copy(d_h.at[pl.ds(r0, _CH)], diA)
            gaA = pltpu.async_copy(a_h.at[siA], raA, sga)
            gbA = pltpu.async_copy(b_h.at[diA], rbA, sgb)
            pltpu.sync_copy(s_h.at[pl.ds(r1, _CH)], siB)
            pltpu.sync_copy(d_h.at[pl.ds(r1, _CH)], diB)
            gaB = pltpu.async_copy(a_h.at[siB], raB, sga)
            gbB = pltpu.async_copy(b_h.at[diB], rbB, sgb)
            gaA.wait()
            s1 = pltpu.async_copy(raA, ga_h.at[pl.ds(r0, _CH)], soa)
            gbA.wait()
            s2 = pltpu.async_copy(rbA, gb_h.at[pl.ds(r0, _CH)], sob)
            gaB.wait()
            s3 = pltpu.async_copy(raB, ga_h.at[pl.ds(r1, _CH)], soa)
            gbB.wait()
            s4 = pltpu.async_copy(rbB, gb_h.at[pl.ds(r1, _CH)], sob)
            s1.wait()
            s2.wait()
            s3.wait()
            s4.wait()
            return carry

        lax.fori_loop(0, pairs, body, 0)

    return gk(a, b, si, di)


def _sc_scatter_add(msg, di, nd1):
    ne, h = msg.shape
    pw = ne // _NW
    pairs = pw // (2 * _CH)
    zr = nd1 // 16
    zsteps = zr // _CH
    zeros_blk = jnp.zeros((_CH, h), jnp.float32)
    mesh = plsc.VectorSubcoreMesh(core_axis_name="c", subcore_axis_name="s")

    @functools.partial(
        pl.kernel, mesh=mesh,
        compiler_params=pltpu.CompilerParams(use_tc_tiling_on_sc=False),
        out_type=jax.ShapeDtypeStruct((2, nd1, h), jnp.float32),
        scratch_types=(pltpu.VMEM((_CH,), jnp.int32),
                       pltpu.VMEM((_CH,), jnp.int32),
                       pltpu.VMEM((_CH, h), jnp.float32),
                       pltpu.VMEM((_CH, h), jnp.float32),
                       pltpu.VMEM((_CH, h), jnp.float32),
                       pltpu.VMEM_SHARED((nd1, h), jnp.float32),
                       pltpu.SemaphoreType.DMA, pltpu.SemaphoreType.DMA,
                       pltpu.SemaphoreType.DMA))
    def sk(m_h, d_h, z_h, out_h, diA, diB, rmA, rmB, zb_v, agg_s,
           sm, ssc, sz):
        cid = lax.axis_index("c")
        sid = lax.axis_index("s")
        wid = sid * 2 + cid
        pltpu.sync_copy(z_h, zb_v)
        zds = [pltpu.async_copy(zb_v, agg_s.at[pl.ds(sid * zr + t * _CH, _CH)],
                                sz) for t in range(zsteps)]
        for dsc in zds:
            dsc.wait()
        plsc.subcore_barrier()
        base = wid * pw

        def body(p, carry):
            r0 = base + p * 2 * _CH
            r1 = r0 + _CH
            ldA = pltpu.async_copy(m_h.at[pl.ds(r0, _CH)], rmA, sm)
            ldB = pltpu.async_copy(m_h.at[pl.ds(r1, _CH)], rmB, sm)
            pltpu.sync_copy(d_h.at[pl.ds(r0, _CH)], diA)
            pltpu.sync_copy(d_h.at[pl.ds(r1, _CH)], diB)
            ldA.wait()
            scA = pltpu.async_copy(rmA, agg_s.at[diA], ssc, add=True)
            ldB.wait()
            scB = pltpu.async_copy(rmB, agg_s.at[diB], ssc, add=True)
            scA.wait()
            scB.wait()
            return carry

        lax.fori_loop(0, pairs, body, 0)
        plsc.subcore_barrier()
        ods = [pltpu.async_copy(agg_s.at[pl.ds(sid * zr + t * _CH, _CH)],
                                out_h.at[cid, pl.ds(sid * zr + t * _CH, _CH)],
                                sz) for t in range(zsteps)]
        for dsc in ods:
            dsc.wait()

    return sk(msg, di, zeros_blk)


def _prep_edges(ei, ns_u, nd_u, bs, h):
    src, dst = ei[0], ei[1]
    srcb = jnp.concatenate([src + i * ns_u for i in range(bs)])
    dstb = jnp.concatenate([dst + i * nd_u for i in range(bs)])
    ne = srcb.shape[0]
    ne_pad = _rup(ne, 2 * _NW * _CH)
    pad = ne_pad - ne
    si = jnp.pad(srcb, (0, pad))
    dg = jnp.pad(dstb, (0, pad))
    ds = jnp.pad(dstb, (0, pad), constant_values=nd_u * bs)
    nd1 = _rup(nd_u * bs + 1, 16 * _CH)
    return si, dg, ds, ne_pad, nd1


def _pad_rows(x, n):
    return jnp.pad(x, ((0, n - x.shape[0]), (0, 0)))


def _block_k(blk, xs_tab, xd_tab, e, si, dg, ds, nd, nd1, extra=None):
    ag, bg = _sc_gather2(xs_tab, xd_tab, si, dg)
    msg, e_new = _edge_tc(ag, bg, e, blk["edge_mlp"])
    agg2 = _sc_scatter_add(msg, ds, nd1)
    xd_new = _node_tc(xd_tab, agg2[:, :nd, :], blk["node_mlp"], extra)
    return xd_new, e_new


def _hop(mp, pp, x_src, sname, dname, geo, edges, bs):
    nd_u = geo[dname + "_latlons"].shape[0]
    ns_u = geo[sname + "_latlons"].shape[0]
    nd = nd_u * bs

    xs = _mlp_ln_tc(x_src, mp["emb_src"])
    xd = jnp.tile(_mlp_ln_tc(geo[dname + "_latlons"], mp["emb_dst"]), (bs, 1))
    e = jnp.tile(_mlp_ln_tc(geo[sname + "_" + dname + "_edge_attr"],
                            mp["emb_edge"]), (bs, 1))
    h = xd.shape[1]
    si, dg, ds, ne_pad, nd1 = _prep_edges(
        edges[sname + "_" + dname + "_edge_index"], ns_u, nd_u, bs, h)
    e = _pad_rows(e, ne_pad)
    xd, _ = _block_k(mp["blocks"][0], xs, xd, e, si, dg, ds, nd, nd1)
    x_lat = xd

    ep = jnp.tile(_mlp_ln_tc(geo[dname + "_" + dname + "_edge_attr"],
                             pp["emb_edge"]), (bs, 1))
    sip, dgp, dsp, nep_pad, nd1p = _prep_edges(
        edges[dname + "_" + dname + "_edge_index"], nd_u, nd_u, bs, h)
    ep = _pad_rows(ep, nep_pad)
    xp = x_lat
    nblk = len(pp["blocks"])
    for i, blk in enumerate(pp["blocks"]):
        res = x_lat if i == nblk - 1 else None
        xp, ep = _block_k(blk, xp, xp, ep, sip, dgp, dsp, nd, nd1p, extra=res)
    return xp


def kernel(x, params, geo, edges):
    bs, n_era, in_ch = x.shape
    xf = jnp.concatenate(
        [x.reshape(bs * n_era, in_ch), jnp.tile(geo["era_latlons"], (bs, 1))],
        axis=-1)
    cur = xf
    for s, d in (("era", "h33"), ("h33", "h32"), ("h32", "h31"),
                 ("h31", "h30")):
        cur = _hop(params[d + "_mapper"], params[d + "_proc"], cur, s, d,
                   geo, edges, bs)
    return _final_tc(cur, params["final_W1"], params["final_b1"],
                     params["final_W2"], bs)


# speedup vs baseline: 1.3366x; 1.1878x over previous
import functools

import jax
import jax.numpy as jnp
from jax import lax
from jax.experimental import pallas as pl
from jax.experimental.pallas import tpu as pltpu
from jax.experimental.pallas import tpu_sc as plsc

_EPS = 1e-5


def _cdiv(a, b):
    return -(-a // b)


def _rup(a, b):
    return _cdiv(a, b) * b


def _ln(o):
    mu = jnp.mean(o, axis=-1, keepdims=True)
    var = jnp.mean(jnp.square(o - mu), axis=-1, keepdims=True)
    return (o - mu) * lax.rsqrt(var + _EPS)


def _row_spec(br, d):
    return pl.BlockSpec((br, d), lambda i: (i, 0))


def _w_spec(shape):
    return pl.BlockSpec(shape, lambda i: tuple(0 for _ in shape))


def _mlp_ln_tc(x, p):
    n, din = x.shape
    dh = p["W1"].shape[1]
    dout = p["W2"].shape[1]
    br = min(1024, _rup(n, 8))
    grid = _cdiv(n, br)

    def body(x_ref, w1, b1, w2, b2, o_ref):
        h = jax.nn.silu(x_ref[...] @ w1[...] + b1[...])
        o_ref[...] = _ln(h @ w2[...] + b2[...])

    return pl.pallas_call(
        body,
        grid=(grid,),
        in_specs=[_row_spec(br, din), _w_spec((din, dh)), _w_spec((1, dh)),
                  _w_spec((dh, dout)), _w_spec((1, dout))],
        out_specs=_row_spec(br, dout),
        out_shape=jax.ShapeDtypeStruct((n, dout), jnp.float32),
    )(x, p["W1"], p["b1"].reshape(1, dh), p["W2"], p["b2"].reshape(1, dout))


def _edge_tc(ag, bg, e, p):
    n, h = e.shape
    br = min(1024, _rup(n, 8))
    grid = _cdiv(n, br)
    w1 = p["W1"]
    w1s, w1d, w1e = w1[:h], w1[h:2 * h], w1[2 * h:]

    def body(a_ref, b_ref, e_ref, ws, wd, we, b1, w2, b2, msg_ref, enew_ref):
        h1 = jax.nn.silu(a_ref[...] @ ws[...] + b_ref[...] @ wd[...]
                         + e_ref[...] @ we[...] + b1[...])
        m = _ln(h1 @ w2[...] + b2[...])
        msg_ref[...] = m
        enew_ref[...] = e_ref[...] + m

    return pl.pallas_call(
        body,
        grid=(grid,),
        in_specs=[_row_spec(br, h), _row_spec(br, h), _row_spec(br, h),
                  _w_spec((h, h)), _w_spec((h, h)), _w_spec((h, h)),
                  _w_spec((1, h)), _w_spec((h, h)), _w_spec((1, h))],
        out_specs=[_row_spec(br, h), _row_spec(br, h)],
        out_shape=[jax.ShapeDtypeStruct((n, h), jnp.float32),
                   jax.ShapeDtypeStruct((n, h), jnp.float32)],
    )(ag, bg, e, w1s, w1d, w1e, p["b1"].reshape(1, h), p["W2"],
      p["b2"].reshape(1, h))


def _node_tc(xd, agg2, p, extra=None):
    n, h = xd.shape
    br = min(1024, _rup(n, 8))
    grid = _cdiv(n, br)
    v1 = p["W1"]
    v1a, v1b = v1[:h], v1[h:]

    def body_base(x_ref, g_ref, wa, wb, b1, w2, b2):
        agg = g_ref[0] + g_ref[1]
        h1 = jax.nn.silu(x_ref[...] @ wa[...] + agg @ wb[...] + b1[...])
        return x_ref[...] + _ln(h1 @ w2[...] + b2[...])

    if extra is None:
        def body(x_ref, g_ref, wa, wb, b1, w2, b2, o_ref):
            o_ref[...] = body_base(x_ref, g_ref, wa, wb, b1, w2, b2)
        extra_in = []
        extra_spec = []
    else:
        def body(x_ref, g_ref, wa, wb, b1, w2, b2, r_ref, o_ref):
            o_ref[...] = body_base(x_ref, g_ref, wa, wb, b1, w2, b2) + r_ref[...]
        extra_in = [extra]
        extra_spec = [_row_spec(br, h)]

    g_spec = pl.BlockSpec((2, br, h), lambda i: (0, i, 0))
    return pl.pallas_call(
        body,
        grid=(grid,),
        in_specs=[_row_spec(br, h), g_spec, _w_spec((h, h)), _w_spec((h, h)),
                  _w_spec((1, h)), _w_spec((h, h)), _w_spec((1, h))]
                 + extra_spec,
        out_specs=_row_spec(br, h),
        out_shape=jax.ShapeDtypeStruct((n, h), jnp.float32),
    )(xd, agg2, v1a, v1b, p["b1"].reshape(1, h), p["W2"],
      p["b2"].reshape(1, h), *extra_in)


def _final_tc(x30, w1, b1, w2, bs):
    n, h = x30.shape
    per = n // bs
    dh = w1.shape[1]
    sel = (jnp.arange(n)[None, :] // per
           == jnp.arange(bs)[:, None]).astype(jnp.float32) / per

    def body(x_ref, w1_ref, b1_ref, w2_ref, sel_ref, o_ref):
        hh = jax.nn.silu(x_ref[...] @ w1_ref[...] + b1_ref[...])
        o_ref[...] = sel_ref[...] @ (hh @ w2_ref[...])

    return pl.pallas_call(
        body,
        grid=(1,),
        in_specs=[_row_spec(n, h), _w_spec((h, dh)), _w_spec((1, dh)),
                  _w_spec((dh, 1)), _w_spec((bs, n))],
        out_specs=pl.BlockSpec((bs, 1), lambda i: (0, 0)),
        out_shape=jax.ShapeDtypeStruct((bs, 1), jnp.float32),
    )(x30, w1, b1.reshape(1, dh), w2, sel)


_NW = 32
_CH = 128


def _sc_gather2(a, b, si, di):
    ne = si.shape[0]
    h = a.shape[1]
    pw = ne // _NW
    steps = pw // _CH
    mesh = plsc.VectorSubcoreMesh(core_axis_name="c", subcore_axis_name="s")

    @functools.partial(
        pl.kernel, mesh=mesh,
        compiler_params=pltpu.CompilerParams(use_tc_tiling_on_sc=False),
        out_type=(jax.ShapeDtypeStruct((ne, h), jnp.float32),
                  jax.ShapeDtypeStruct((ne, h), jnp.float32)),
        scratch_types=(pltpu.VMEM((_CH,), jnp.int32),
                       pltpu.VMEM((_CH,), jnp.int32),
                       pltpu.VMEM((_CH, h), jnp.float32),
                       pltpu.VMEM((_CH, h), jnp.float32),
                       pltpu.SemaphoreType.DMA,
                       pltpu.SemaphoreType.DMA))
    def gk(a_h, b_h, s_h, d_h, ga_h, gb_h, si_v, di_v, ra_v, rb_v, sa, sb):
        wid = lax.axis_index("s") * 2 + lax.axis_index("c")
        base = wid * pw

        def body(j, carry):
            r0 = base + j * _CH
            pltpu.sync_copy(s_h.at[pl.ds(r0, _CH)], si_v)
            pltpu.sync_copy(d_h.at[pl.ds(r0, _CH)], di_v)
            ca = pltpu.async_copy(a_h.at[si_v], ra_v, sa)
            cb = pltpu.async_copy(b_h.at[di_v], rb_v, sb)
            ca.wait()
            cb.wait()
            pltpu.sync_copy(ra_v, ga_h.at[pl.ds(r0, _CH)])
            pltpu.sync_copy(rb_v, gb_h.at[pl.ds(r0, _CH)])
            return carry

        lax.fori_loop(0, steps, body, 0)

    return gk(a, b, si, di)


def _sc_scatter_add(msg, di, nd1):
    ne, h = msg.shape
    pw = ne // _NW
    steps = pw // _CH
    zr = nd1 // 16
    zsteps = zr // _CH
    zeros_blk = jnp.zeros((_CH, h), jnp.float32)
    mesh = plsc.VectorSubcoreMesh(core_axis_name="c", subcore_axis_name="s")

    @functools.partial(
        pl.kernel, mesh=mesh,
        compiler_params=pltpu.CompilerParams(use_tc_tiling_on_sc=False),
        out_type=jax.ShapeDtypeStruct((2, nd1, h), jnp.float32),
        scratch_types=(pltpu.VMEM((_CH,), jnp.int32),
                       pltpu.VMEM((_CH, h), jnp.float32),
                       pltpu.VMEM((_CH, h), jnp.float32),
                       pltpu.VMEM_SHARED((nd1, h), jnp.float32)))
    def sk(m_h, d_h, z_h, out_h, di_v, rm_v, zb_v, agg_s):
        cid = lax.axis_index("c")
        sid = lax.axis_index("s")
        wid = sid * 2 + cid
        pltpu.sync_copy(z_h, zb_v)

        def zbody(j, carry):
            pltpu.sync_copy(zb_v, agg_s.at[pl.ds(sid * zr + j * _CH, _CH)])
            return carry

        lax.fori_loop(0, zsteps, zbody, 0)
        plsc.subcore_barrier()
        base = wid * pw

        def body(j, carry):
            r0 = base + j * _CH
            pltpu.sync_copy(d_h.at[pl.ds(r0, _CH)], di_v)
            pltpu.sync_copy(m_h.at[pl.ds(r0, _CH)], rm_v)
            pltpu.sync_copy(rm_v, agg_s.at[di_v], add=True)
            return carry

        lax.fori_loop(0, steps, body, 0)
        plsc.subcore_barrier()

        def obody(j, carry):
            r0 = sid * zr + j * _CH
            pltpu.sync_copy(agg_s.at[pl.ds(r0, _CH)],
                            out_h.at[cid, pl.ds(r0, _CH)])
            return carry

        lax.fori_loop(0, zsteps, obody, 0)

    return sk(msg, di, zeros_blk)


def _prep_edges(ei, ns_u, nd_u, bs, h):
    src, dst = ei[0], ei[1]
    srcb = jnp.concatenate([src + i * ns_u for i in range(bs)])
    dstb = jnp.concatenate([dst + i * nd_u for i in range(bs)])
    ne = srcb.shape[0]
    ne_pad = _rup(ne, _NW * _CH)
    pad = ne_pad - ne
    si = jnp.pad(srcb, (0, pad))
    dg = jnp.pad(dstb, (0, pad))
    ds = jnp.pad(dstb, (0, pad), constant_values=nd_u * bs)
    nd1 = _rup(nd_u * bs + 1, 16 * _CH)
    return si, dg, ds, ne_pad, nd1


def _pad_rows(x, n):
    return jnp.pad(x, ((0, n - x.shape[0]), (0, 0)))


def _block_k(blk, xs_tab, xd_tab, e, si, dg, ds, nd, nd1, extra=None):
    ag, bg = _sc_gather2(xs_tab, xd_tab, si, dg)
    msg, e_new = _edge_tc(ag, bg, e, blk["edge_mlp"])
    agg2 = _sc_scatter_add(msg, ds, nd1)
    xd_new = _node_tc(xd_tab, agg2[:, :nd, :], blk["node_mlp"], extra)
    return xd_new, e_new


def _hop(mp, pp, x_src, sname, dname, geo, edges, bs):
    nd_u = geo[dname + "_latlons"].shape[0]
    ns_u = geo[sname + "_latlons"].shape[0]
    nd = nd_u * bs

    xs = _mlp_ln_tc(x_src, mp["emb_src"])
    xd = jnp.tile(_mlp_ln_tc(geo[dname + "_latlons"], mp["emb_dst"]), (bs, 1))
    e = jnp.tile(_mlp_ln_tc(geo[sname + "_" + dname + "_edge_attr"],
                            mp["emb_edge"]), (bs, 1))
    h = xd.shape[1]
    si, dg, ds, ne_pad, nd1 = _prep_edges(
        edges[sname + "_" + dname + "_edge_index"], ns_u, nd_u, bs, h)
    e = _pad_rows(e, ne_pad)
    xd, _ = _block_k(mp["blocks"][0], xs, xd, e, si, dg, ds, nd, nd1)
    x_lat = xd

    ep = jnp.tile(_mlp_ln_tc(geo[dname + "_" + dname + "_edge_attr"],
                             pp["emb_edge"]), (bs, 1))
    sip, dgp, dsp, nep_pad, nd1p = _prep_edges(
        edges[dname + "_" + dname + "_edge_index"], nd_u, nd_u, bs, h)
    ep = _pad_rows(ep, nep_pad)
    xp = x_lat
    nblk = len(pp["blocks"])
    for i, blk in enumerate(pp["blocks"]):
        res = x_lat if i == nblk - 1 else None
        xp, ep = _block_k(blk, xp, xp, ep, sip, dgp, dsp, nd, nd1p, extra=res)
    return xp


def kernel(x, params, geo, edges):
    bs, n_era, in_ch = x.shape
    xf = jnp.concatenate(
        [x.reshape(bs * n_era, in_ch), jnp.tile(geo["era_latlons"], (bs, 1))],
        axis=-1)
    cur = xf
    for s, d in (("era", "h33"), ("h33", "h32"), ("h32", "h31"),
                 ("h31", "h30")):
        cur = _hop(params[d + "_mapper"], params[d + "_proc"], cur, s, d,
                   geo, edges, bs)
    return _final_tc(cur, params["final_W1"], params["final_b1"],
                     params["final_W2"], bs)


# speedup vs baseline: 1.3572x; 1.0155x over previous
import functools

import jax
import jax.numpy as jnp
from jax import lax
from jax.experimental import pallas as pl
from jax.experimental.pallas import tpu as pltpu
from jax.experimental.pallas import tpu_sc as plsc

_EPS = 1e-5


def _cdiv(a, b):
    return -(-a // b)


def _rup(a, b):
    return _cdiv(a, b) * b


def _ln(o):
    mu = jnp.mean(o, axis=-1, keepdims=True)
    var = jnp.mean(jnp.square(o - mu), axis=-1, keepdims=True)
    return (o - mu) * lax.rsqrt(var + _EPS)


def _row_spec(br, d):
    return pl.BlockSpec((br, d), lambda i: (i, 0))


def _w_spec(shape):
    return pl.BlockSpec(shape, lambda i: tuple(0 for _ in shape))


def _mlp_ln_tc(x, p):
    n, din = x.shape
    dh = p["W1"].shape[1]
    dout = p["W2"].shape[1]
    br = min(1024, _rup(n, 8))
    grid = _cdiv(n, br)

    def body(x_ref, w1, b1, w2, b2, o_ref):
        h = jax.nn.silu(x_ref[...] @ w1[...] + b1[...])
        o_ref[...] = _ln(h @ w2[...] + b2[...])

    return pl.pallas_call(
        body,
        grid=(grid,),
        in_specs=[_row_spec(br, din), _w_spec((din, dh)), _w_spec((1, dh)),
                  _w_spec((dh, dout)), _w_spec((1, dout))],
        out_specs=_row_spec(br, dout),
        out_shape=jax.ShapeDtypeStruct((n, dout), jnp.float32),
    )(x, p["W1"], p["b1"].reshape(1, dh), p["W2"], p["b2"].reshape(1, dout))


def _edge_tc(ag, bg, e, p):
    n, h = e.shape
    br = min(1024, _rup(n, 8))
    grid = _cdiv(n, br)
    w1 = p["W1"]
    w1s, w1d, w1e = w1[:h], w1[h:2 * h], w1[2 * h:]

    def body(a_ref, b_ref, e_ref, ws, wd, we, b1, w2, b2, msg_ref, enew_ref):
        h1 = jax.nn.silu(a_ref[...] @ ws[...] + b_ref[...] @ wd[...]
                         + e_ref[...] @ we[...] + b1[...])
        m = _ln(h1 @ w2[...] + b2[...])
        msg_ref[...] = m
        enew_ref[...] = e_ref[...] + m

    return pl.pallas_call(
        body,
        grid=(grid,),
        in_specs=[_row_spec(br, h), _row_spec(br, h), _row_spec(br, h),
                  _w_spec((h, h)), _w_spec((h, h)), _w_spec((h, h)),
                  _w_spec((1, h)), _w_spec((h, h)), _w_spec((1, h))],
        out_specs=[_row_spec(br, h), _row_spec(br, h)],
        out_shape=[jax.ShapeDtypeStruct((n, h), jnp.float32),
                   jax.ShapeDtypeStruct((n, h), jnp.float32)],
    )(ag, bg, e, w1s, w1d, w1e, p["b1"].reshape(1, h), p["W2"],
      p["b2"].reshape(1, h))


def _node_tc(xd, agg2, p, extra=None):
    n, h = xd.shape
    br = min(1024, _rup(n, 8))
    grid = _cdiv(n, br)
    v1 = p["W1"]
    v1a, v1b = v1[:h], v1[h:]

    def body_base(x_ref, g_ref, wa, wb, b1, w2, b2):
        agg = g_ref[0] + g_ref[1]
        h1 = jax.nn.silu(x_ref[...] @ wa[...] + agg @ wb[...] + b1[...])
        return x_ref[...] + _ln(h1 @ w2[...] + b2[...])

    if extra is None:
        def body(x_ref, g_ref, wa, wb, b1, w2, b2, o_ref):
            o_ref[...] = body_base(x_ref, g_ref, wa, wb, b1, w2, b2)
        extra_in = []
        extra_spec = []
    else:
        def body(x_ref, g_ref, wa, wb, b1, w2, b2, r_ref, o_ref):
            o_ref[...] = body_base(x_ref, g_ref, wa, wb, b1, w2, b2) + r_ref[...]
        extra_in = [extra]
        extra_spec = [_row_spec(br, h)]

    g_spec = pl.BlockSpec((2, br, h), lambda i: (0, i, 0))
    return pl.pallas_call(
        body,
        grid=(grid,),
        in_specs=[_row_spec(br, h), g_spec, _w_spec((h, h)), _w_spec((h, h)),
                  _w_spec((1, h)), _w_spec((h, h)), _w_spec((1, h))]
                 + extra_spec,
        out_specs=_row_spec(br, h),
        out_shape=jax.ShapeDtypeStruct((n, h), jnp.float32),
    )(xd, agg2, v1a, v1b, p["b1"].reshape(1, h), p["W2"],
      p["b2"].reshape(1, h), *extra_in)


def _final_tc(x30, w1, b1, w2, bs):
    n, h = x30.shape
    per = n // bs
    dh = w1.shape[1]
    sel = (jnp.arange(n)[None, :] // per
           == jnp.arange(bs)[:, None]).astype(jnp.float32) / per

    def body(x_ref, w1_ref, b1_ref, w2_ref, sel_ref, o_ref):
        hh = jax.nn.silu(x_ref[...] @ w1_ref[...] + b1_ref[...])
        o_ref[...] = sel_ref[...] @ (hh @ w2_ref[...])

    return pl.pallas_call(
        body,
        grid=(1,),
        in_specs=[_row_spec(n, h), _w_spec((h, dh)), _w_spec((1, dh)),
                  _w_spec((dh, 1)), _w_spec((bs, n))],
        out_specs=pl.BlockSpec((bs, 1), lambda i: (0, 0)),
        out_shape=jax.ShapeDtypeStruct((bs, 1), jnp.float32),
    )(x30, w1, b1.reshape(1, dh), w2, sel)


_NW = 32
_CH = 128


def _sc_gather2(a, b, si, di):
    ne = si.shape[0]
    h = a.shape[1]
    pw = ne // _NW
    steps = pw // _CH
    mesh = plsc.VectorSubcoreMesh(core_axis_name="c", subcore_axis_name="s")

    @functools.partial(
        pl.kernel, mesh=mesh,
        compiler_params=pltpu.CompilerParams(use_tc_tiling_on_sc=False),
        out_type=(jax.ShapeDtypeStruct((ne, h), jnp.float32),
                  jax.ShapeDtypeStruct((ne, h), jnp.float32)),
        scratch_types=(pltpu.VMEM((_CH,), jnp.int32),
                       pltpu.VMEM((_CH,), jnp.int32),
                       pltpu.VMEM((_CH, h), jnp.float32),
                       pltpu.VMEM((_CH, h), jnp.float32),
                       pltpu.SemaphoreType.DMA, pltpu.SemaphoreType.DMA,
                       pltpu.SemaphoreType.DMA, pltpu.SemaphoreType.DMA))
    def gk(a_h, b_h, s_h, d_h, ga_h, gb_h, si_v, di_v, ra_v, rb_v,
           sa, sb, sia, sib):
        wid = lax.axis_index("s") * 2 + lax.axis_index("c")
        base = wid * pw

        def body(j, carry):
            r0 = base + j * _CH
            ci = pltpu.async_copy(s_h.at[pl.ds(r0, _CH)], si_v, sia)
            cd = pltpu.async_copy(d_h.at[pl.ds(r0, _CH)], di_v, sib)
            ci.wait()
            cd.wait()
            ca = pltpu.async_copy(a_h.at[si_v], ra_v, sa)
            cb = pltpu.async_copy(b_h.at[di_v], rb_v, sb)
            ca.wait()
            cb.wait()
            pltpu.sync_copy(ra_v, ga_h.at[pl.ds(r0, _CH)])
            pltpu.sync_copy(rb_v, gb_h.at[pl.ds(r0, _CH)])
            return carry

        lax.fori_loop(0, steps, body, 0)

    return gk(a, b, si, di)


def _sc_scatter_add(msg, di, nd1):
    ne, h = msg.shape
    pw = ne // _NW
    steps = pw // _CH
    zr = nd1 // 16
    zsteps = zr // _CH
    zeros_blk = jnp.zeros((_CH, h), jnp.float32)
    mesh = plsc.VectorSubcoreMesh(core_axis_name="c", subcore_axis_name="s")

    @functools.partial(
        pl.kernel, mesh=mesh,
        compiler_params=pltpu.CompilerParams(use_tc_tiling_on_sc=False),
        out_type=jax.ShapeDtypeStruct((2, nd1, h), jnp.float32),
        scratch_types=(pltpu.VMEM((_CH,), jnp.int32),
                       pltpu.VMEM((_CH, h), jnp.float32),
                       pltpu.VMEM((_CH, h), jnp.float32),
                       pltpu.VMEM_SHARED((nd1, h), jnp.float32),
                       pltpu.SemaphoreType.DMA, pltpu.SemaphoreType.DMA))
    def sk(m_h, d_h, z_h, out_h, di_v, rm_v, zb_v, agg_s, sdi, smv):
        cid = lax.axis_index("c")
        sid = lax.axis_index("s")
        wid = sid * 2 + cid
        pltpu.sync_copy(z_h, zb_v)

        def zbody(j, carry):
            pltpu.sync_copy(zb_v, agg_s.at[pl.ds(sid * zr + j * _CH, _CH)])
            return carry

        lax.fori_loop(0, zsteps, zbody, 0)
        plsc.subcore_barrier()
        base = wid * pw

        def body(j, carry):
            r0 = base + j * _CH
            c1 = pltpu.async_copy(d_h.at[pl.ds(r0, _CH)], di_v, sdi)
            c2 = pltpu.async_copy(m_h.at[pl.ds(r0, _CH)], rm_v, smv)
            c1.wait()
            c2.wait()
            pltpu.sync_copy(rm_v, agg_s.at[di_v], add=True)
            return carry

        lax.fori_loop(0, steps, body, 0)
        plsc.subcore_barrier()

        def obody(j, carry):
            r0 = sid * zr + j * _CH
            pltpu.sync_copy(agg_s.at[pl.ds(r0, _CH)],
                            out_h.at[cid, pl.ds(r0, _CH)])
            return carry

        lax.fori_loop(0, zsteps, obody, 0)

    return sk(msg, di, zeros_blk)


def _prep_edges(ei, ns_u, nd_u, bs, h):
    src, dst = ei[0], ei[1]
    srcb = jnp.concatenate([src + i * ns_u for i in range(bs)])
    dstb = jnp.concatenate([dst + i * nd_u for i in range(bs)])
    ne = srcb.shape[0]
    ne_pad = _rup(ne, _NW * _CH)
    pad = ne_pad - ne
    si = jnp.pad(srcb, (0, pad))
    dg = jnp.pad(dstb, (0, pad))
    ds = jnp.pad(dstb, (0, pad), constant_values=nd_u * bs)
    nd1 = _rup(nd_u * bs + 1, 16 * _CH)
    return si, dg, ds, ne_pad, nd1


def _pad_rows(x, n):
    return jnp.pad(x, ((0, n - x.shape[0]), (0, 0)))


def _block_k(blk, xs_tab, xd_tab, e, si, dg, ds, nd, nd1, extra=None):
    ag, bg = _sc_gather2(xs_tab, xd_tab, si, dg)
    msg, e_new = _edge_tc(ag, bg, e, blk["edge_mlp"])
    agg2 = _sc_scatter_add(msg, ds, nd1)
    xd_new = _node_tc(xd_tab, agg2[:, :nd, :], blk["node_mlp"], extra)
    return xd_new, e_new


def _hop(mp, pp, x_src, sname, dname, geo, edges, bs):
    nd_u = geo[dname + "_latlons"].shape[0]
    ns_u = geo[sname + "_latlons"].shape[0]
    nd = nd_u * bs

    xs = _mlp_ln_tc(x_src, mp["emb_src"])
    xd = jnp.tile(_mlp_ln_tc(geo[dname + "_latlons"], mp["emb_dst"]), (bs, 1))
    e = jnp.tile(_mlp_ln_tc(geo[sname + "_" + dname + "_edge_attr"],
                            mp["emb_edge"]), (bs, 1))
    h = xd.shape[1]
    si, dg, ds, ne_pad, nd1 = _prep_edges(
        edges[sname + "_" + dname + "_edge_index"], ns_u, nd_u, bs, h)
    e = _pad_rows(e, ne_pad)
    xd, _ = _block_k(mp["blocks"][0], xs, xd, e, si, dg, ds, nd, nd1)
    x_lat = xd

    ep = jnp.tile(_mlp_ln_tc(geo[dname + "_" + dname + "_edge_attr"],
                             pp["emb_edge"]), (bs, 1))
    sip, dgp, dsp, nep_pad, nd1p = _prep_edges(
        edges[dname + "_" + dname + "_edge_index"], nd_u, nd_u, bs, h)
    ep = _pad_rows(ep, nep_pad)
    xp = x_lat
    nblk = len(pp["blocks"])
    for i, blk in enumerate(pp["blocks"]):
        res = x_lat if i == nblk - 1 else None
        xp, ep = _block_k(blk, xp, xp, ep, sip, dgp, dsp, nd, nd1p, extra=res)
    return xp


def kernel(x, params, geo, edges):
    bs, n_era, in_ch = x.shape
    xf = jnp.concatenate(
        [x.reshape(bs * n_era, in_ch), jnp.tile(geo["era_latlons"], (bs, 1))],
        axis=-1)
    cur = xf
    for s, d in (("era", "h33"), ("h33", "h32"), ("h32", "h31"),
                 ("h31", "h30")):
        cur = _hop(params[d + "_mapper"], params[d + "_proc"], cur, s, d,
                   geo, edges, bs)
    return _final_tc(cur, params["final_W1"], params["final_b1"],
                     params["final_W2"], bs)


# speedup vs baseline: 1.3591x; 1.0013x over previous
import functools

import jax
import jax.numpy as jnp
from jax import lax
from jax.experimental import pallas as pl
from jax.experimental.pallas import tpu as pltpu
from jax.experimental.pallas import tpu_sc as plsc

_EPS = 1e-5


def _cdiv(a, b):
    return -(-a // b)


def _rup(a, b):
    return _cdiv(a, b) * b


def _ln(o):
    mu = jnp.mean(o, axis=-1, keepdims=True)
    var = jnp.mean(jnp.square(o - mu), axis=-1, keepdims=True)
    return (o - mu) * lax.rsqrt(var + _EPS)


def _row_spec(br, d):
    return pl.BlockSpec((br, d), lambda i: (i, 0))


def _w_spec(shape):
    return pl.BlockSpec(shape, lambda i: tuple(0 for _ in shape))


def _mlp_ln_tc(x, p):
    n, din = x.shape
    dh = p["W1"].shape[1]
    dout = p["W2"].shape[1]
    br = min(1024, _rup(n, 8))
    grid = _cdiv(n, br)

    def body(x_ref, w1, b1, w2, b2, o_ref):
        h = jax.nn.silu(x_ref[...] @ w1[...] + b1[...])
        o_ref[...] = _ln(h @ w2[...] + b2[...])

    return pl.pallas_call(
        body,
        grid=(grid,),
        in_specs=[_row_spec(br, din), _w_spec((din, dh)), _w_spec((1, dh)),
                  _w_spec((dh, dout)), _w_spec((1, dout))],
        out_specs=_row_spec(br, dout),
        out_shape=jax.ShapeDtypeStruct((n, dout), jnp.float32),
    )(x, p["W1"], p["b1"].reshape(1, dh), p["W2"], p["b2"].reshape(1, dout))


def _edge_tc(ag, bg, e, p):
    n, h = e.shape
    br = min(1024, _rup(n, 8))
    grid = _cdiv(n, br)
    w1 = p["W1"]
    w1s, w1d, w1e = w1[:h], w1[h:2 * h], w1[2 * h:]

    def body(a_ref, b_ref, e_ref, ws, wd, we, b1, w2, b2, msg_ref, enew_ref):
        h1 = jax.nn.silu(a_ref[...] @ ws[...] + b_ref[...] @ wd[...]
                         + e_ref[...] @ we[...] + b1[...])
        m = _ln(h1 @ w2[...] + b2[...])
        msg_ref[...] = m
        enew_ref[...] = e_ref[...] + m

    return pl.pallas_call(
        body,
        grid=(grid,),
        in_specs=[_row_spec(br, h), _row_spec(br, h), _row_spec(br, h),
                  _w_spec((h, h)), _w_spec((h, h)), _w_spec((h, h)),
                  _w_spec((1, h)), _w_spec((h, h)), _w_spec((1, h))],
        out_specs=[_row_spec(br, h), _row_spec(br, h)],
        out_shape=[jax.ShapeDtypeStruct((n, h), jnp.float32),
                   jax.ShapeDtypeStruct((n, h), jnp.float32)],
    )(ag, bg, e, w1s, w1d, w1e, p["b1"].reshape(1, h), p["W2"],
      p["b2"].reshape(1, h))


def _node_tc(xd, agg2, p, extra=None):
    n, h = xd.shape
    br = min(1024, _rup(n, 8))
    grid = _cdiv(n, br)
    v1 = p["W1"]
    v1a, v1b = v1[:h], v1[h:]

    def body_base(x_ref, g_ref, wa, wb, b1, w2, b2):
        agg = g_ref[0] + g_ref[1]
        h1 = jax.nn.silu(x_ref[...] @ wa[...] + agg @ wb[...] + b1[...])
        return x_ref[...] + _ln(h1 @ w2[...] + b2[...])

    if extra is None:
        def body(x_ref, g_ref, wa, wb, b1, w2, b2, o_ref):
            o_ref[...] = body_base(x_ref, g_ref, wa, wb, b1, w2, b2)
        extra_in = []
        extra_spec = []
    else:
        def body(x_ref, g_ref, wa, wb, b1, w2, b2, r_ref, o_ref):
            o_ref[...] = body_base(x_ref, g_ref, wa, wb, b1, w2, b2) + r_ref[...]
        extra_in = [extra]
        extra_spec = [_row_spec(br, h)]

    g_spec = pl.BlockSpec((2, br, h), lambda i: (0, i, 0))
    return pl.pallas_call(
        body,
        grid=(grid,),
        in_specs=[_row_spec(br, h), g_spec, _w_spec((h, h)), _w_spec((h, h)),
                  _w_spec((1, h)), _w_spec((h, h)), _w_spec((1, h))]
                 + extra_spec,
        out_specs=_row_spec(br, h),
        out_shape=jax.ShapeDtypeStruct((n, h), jnp.float32),
    )(xd, agg2, v1a, v1b, p["b1"].reshape(1, h), p["W2"],
      p["b2"].reshape(1, h), *extra_in)


def _final_tc(x30, w1, b1, w2, bs):
    n, h = x30.shape
    per = n // bs
    dh = w1.shape[1]
    sel = (jnp.arange(n)[None, :] // per
           == jnp.arange(bs)[:, None]).astype(jnp.float32) / per

    def body(x_ref, w1_ref, b1_ref, w2_ref, sel_ref, o_ref):
        hh = jax.nn.silu(x_ref[...] @ w1_ref[...] + b1_ref[...])
        o_ref[...] = sel_ref[...] @ (hh @ w2_ref[...])

    return pl.pallas_call(
        body,
        grid=(1,),
        in_specs=[_row_spec(n, h), _w_spec((h, dh)), _w_spec((1, dh)),
                  _w_spec((dh, 1)), _w_spec((bs, n))],
        out_specs=pl.BlockSpec((bs, 1), lambda i: (0, 0)),
        out_shape=jax.ShapeDtypeStruct((bs, 1), jnp.float32),
    )(x30, w1, b1.reshape(1, dh), w2, sel)


_NW = 32
_CH = 128


def _sc_gather2(a, b, si, di):
    ne = si.shape[0]
    h = a.shape[1]
    pw = ne // _NW
    steps = pw // _CH
    mesh = plsc.VectorSubcoreMesh(core_axis_name="c", subcore_axis_name="s")

    @functools.partial(
        pl.kernel, mesh=mesh,
        compiler_params=pltpu.CompilerParams(use_tc_tiling_on_sc=False),
        out_type=(jax.ShapeDtypeStruct((ne, h), jnp.float32),
                  jax.ShapeDtypeStruct((ne, h), jnp.float32)),
        scratch_types=(pltpu.VMEM((_CH,), jnp.int32),
                       pltpu.VMEM((_CH,), jnp.int32),
                       pltpu.VMEM((_CH, h), jnp.float32),
                       pltpu.VMEM((_CH, h), jnp.float32),
                       pltpu.SemaphoreType.DMA, pltpu.SemaphoreType.DMA,
                       pltpu.SemaphoreType.DMA, pltpu.SemaphoreType.DMA))
    def gk(a_h, b_h, s_h, d_h, ga_h, gb_h, si_v, di_v, ra_v, rb_v,
           sa, sb, sia, sib):
        wid = lax.axis_index("s") * 2 + lax.axis_index("c")
        base = wid * pw

        def body(j, carry):
            r0 = base + j * _CH
            ci = pltpu.async_copy(s_h.at[pl.ds(r0, _CH)], si_v, sia)
            cd = pltpu.async_copy(d_h.at[pl.ds(r0, _CH)], di_v, sib)
            ci.wait()
            cd.wait()
            ca = pltpu.async_copy(a_h.at[si_v], ra_v, sa)
            cb = pltpu.async_copy(b_h.at[di_v], rb_v, sb)
            ca.wait()
            so1 = pltpu.async_copy(ra_v, ga_h.at[pl.ds(r0, _CH)], sia)
            cb.wait()
            so2 = pltpu.async_copy(rb_v, gb_h.at[pl.ds(r0, _CH)], sib)
            so1.wait()
            so2.wait()
            return carry

        lax.fori_loop(0, steps, body, 0)

    return gk(a, b, si, di)


def _sc_scatter_add(msg, di, nd1):
    ne, h = msg.shape
    pw = ne // _NW
    steps = pw // _CH
    zr = nd1 // 16
    zsteps = zr // _CH
    zeros_blk = jnp.zeros((_CH, h), jnp.float32)
    mesh = plsc.VectorSubcoreMesh(core_axis_name="c", subcore_axis_name="s")

    @functools.partial(
        pl.kernel, mesh=mesh,
        compiler_params=pltpu.CompilerParams(use_tc_tiling_on_sc=False),
        out_type=jax.ShapeDtypeStruct((2, nd1, h), jnp.float32),
        scratch_types=(pltpu.VMEM((_CH,), jnp.int32),
                       pltpu.VMEM((_CH, h), jnp.float32),
                       pltpu.VMEM((_CH, h), jnp.float32),
                       pltpu.VMEM_SHARED((nd1, h), jnp.float32),
                       pltpu.SemaphoreType.DMA, pltpu.SemaphoreType.DMA))
    def sk(m_h, d_h, z_h, out_h, di_v, rm_v, zb_v, agg_s, sdi, smv):
        cid = lax.axis_index("c")
        sid = lax.axis_index("s")
        wid = sid * 2 + cid
        pltpu.sync_copy(z_h, zb_v)

        def zbody(j, carry):
            pltpu.sync_copy(zb_v, agg_s.at[pl.ds(sid * zr + j * _CH, _CH)])
            return carry

        lax.fori_loop(0, zsteps, zbody, 0)
        plsc.subcore_barrier()
        base = wid * pw

        def body(j, carry):
            r0 = base + j * _CH
            c1 = pltpu.async_copy(d_h.at[pl.ds(r0, _CH)], di_v, sdi)
            c2 = pltpu.async_copy(m_h.at[pl.ds(r0, _CH)], rm_v, smv)
            c1.wait()
            c2.wait()
            pltpu.sync_copy(rm_v, agg_s.at[di_v], add=True)
            return carry

        lax.fori_loop(0, steps, body, 0)
        plsc.subcore_barrier()

        def obody(j, carry):
            r0 = sid * zr + j * _CH
            pltpu.sync_copy(agg_s.at[pl.ds(r0, _CH)],
                            out_h.at[cid, pl.ds(r0, _CH)])
            return carry

        lax.fori_loop(0, zsteps, obody, 0)

    return sk(msg, di, zeros_blk)


def _prep_edges(ei, ns_u, nd_u, bs, h):
    src, dst = ei[0], ei[1]
    srcb = jnp.concatenate([src + i * ns_u for i in range(bs)])
    dstb = jnp.concatenate([dst + i * nd_u for i in range(bs)])
    ne = srcb.shape[0]
    ne_pad = _rup(ne, _NW * _CH)
    pad = ne_pad - ne
    si = jnp.pad(srcb, (0, pad))
    dg = jnp.pad(dstb, (0, pad))
    ds = jnp.pad(dstb, (0, pad), constant_values=nd_u * bs)
    nd1 = _rup(nd_u * bs + 1, 16 * _CH)
    return si, dg, ds, ne_pad, nd1


def _pad_rows(x, n):
    return jnp.pad(x, ((0, n - x.shape[0]), (0, 0)))


def _block_k(blk, xs_tab, xd_tab, e, si, dg, ds, nd, nd1, extra=None):
    ag, bg = _sc_gather2(xs_tab, xd_tab, si, dg)
    msg, e_new = _edge_tc(ag, bg, e, blk["edge_mlp"])
    agg2 = _sc_scatter_add(msg, ds, nd1)
    xd_new = _node_tc(xd_tab, agg2[:, :nd, :], blk["node_mlp"], extra)
    return xd_new, e_new


def _hop(mp, pp, x_src, sname, dname, geo, edges, bs):
    nd_u = geo[dname + "_latlons"].shape[0]
    ns_u = geo[sname + "_latlons"].shape[0]
    nd = nd_u * bs

    xs = _mlp_ln_tc(x_src, mp["emb_src"])
    xd = jnp.tile(_mlp_ln_tc(geo[dname + "_latlons"], mp["emb_dst"]), (bs, 1))
    e = jnp.tile(_mlp_ln_tc(geo[sname + "_" + dname + "_edge_attr"],
                            mp["emb_edge"]), (bs, 1))
    h = xd.shape[1]
    si, dg, ds, ne_pad, nd1 = _prep_edges(
        edges[sname + "_" + dname + "_edge_index"], ns_u, nd_u, bs, h)
    e = _pad_rows(e, ne_pad)
    xd, _ = _block_k(mp["blocks"][0], xs, xd, e, si, dg, ds, nd, nd1)
    x_lat = xd

    ep = jnp.tile(_mlp_ln_tc(geo[dname + "_" + dname + "_edge_attr"],
                             pp["emb_edge"]), (bs, 1))
    sip, dgp, dsp, nep_pad, nd1p = _prep_edges(
        edges[dname + "_" + dname + "_edge_index"], nd_u, nd_u, bs, h)
    ep = _pad_rows(ep, nep_pad)
    xp = x_lat
    nblk = len(pp["blocks"])
    for i, blk in enumerate(pp["blocks"]):
        res = x_lat if i == nblk - 1 else None
        xp, ep = _block_k(blk, xp, xp, ep, sip, dgp, dsp, nd, nd1p, extra=res)
    return xp


def kernel(x, params, geo, edges):
    bs, n_era, in_ch = x.shape
    xf = jnp.concatenate(
        [x.reshape(bs * n_era, in_ch), jnp.tile(geo["era_latlons"], (bs, 1))],
        axis=-1)
    cur = xf
    for s, d in (("era", "h33"), ("h33", "h32"), ("h32", "h31"),
                 ("h31", "h30")):
        cur = _hop(params[d + "_mapper"], params[d + "_proc"], cur, s, d,
                   geo, edges, bs)
    return _final_tc(cur, params["final_W1"], params["final_b1"],
                     params["final_W2"], bs)
